# baseline (device time: 24665 ns/iter reference)
import jax
import jax.numpy as jnp
from jax import lax
from jax.experimental import pallas as pl
from jax.experimental.pallas import tpu as pltpu

B, SQ, D = 2, 128, 512
HQ_LOCAL = 8
DH = 64
GQ = 4
SCALE = 0.125


def kernel(x, Wq, Wo, Wk, Wv):
    def body(x_ref, wq_ref, wo_ref, wk_ref, wv_ref, out_ref,
             send_ref, recv_ref, send_sems, recv_sems):
        my_pos = lax.axis_index("i")
        p1 = my_pos ^ 1
        p2 = 3 - my_pos

        barrier_sem = pltpu.get_barrier_semaphore()
        for nbr in (p1, p2):
            pl.semaphore_signal(barrier_sem, inc=1, device_id=(nbr,),
                                device_id_type=pl.DeviceIdType.MESH)
        pl.semaphore_wait(barrier_sem, 2)

        wq = wq_ref[:, :].astype(jnp.bfloat16)
        wo = wo_ref[:, :].astype(jnp.bfloat16)
        wk = wk_ref[:, pl.ds(my_pos * 2 * DH, 2 * DH)].astype(jnp.bfloat16)
        wv = wv_ref[:, pl.ds(my_pos * 2 * DH, 2 * DH)].astype(jnp.bfloat16)

        partials = []
        for b in range(B):
            xb = x_ref[b].astype(jnp.bfloat16)
            q = jnp.dot(xb, wq, preferred_element_type=jnp.float32)
            k = jnp.dot(xb, wk, preferred_element_type=jnp.float32)
            v = jnp.dot(xb, wv, preferred_element_type=jnp.float32)
            k = k.astype(jnp.bfloat16)
            v = v.astype(jnp.bfloat16)
            outs = []
            for h in range(HQ_LOCAL):
                qh = q[:, h * DH:(h + 1) * DH].astype(jnp.bfloat16)
                g = h // GQ
                kh = k[:, g * DH:(g + 1) * DH]
                vh = v[:, g * DH:(g + 1) * DH]
                s = lax.dot_general(
                    qh, kh, (((1,), (1,)), ((), ())),
                    preferred_element_type=jnp.float32) * SCALE
                m = jnp.max(s, axis=-1, keepdims=True)
                p = jnp.exp(s - m)
                l = jnp.sum(p, axis=-1, keepdims=True)
                pn = (p / l).astype(jnp.bfloat16)
                outs.append(jnp.dot(pn, vh, preferred_element_type=jnp.float32))
            o = jnp.concatenate(outs, axis=-1).astype(jnp.bfloat16)
            partials.append(jnp.dot(o, wo, preferred_element_type=jnp.float32))

        for b in range(B):
            send_ref[b] = partials[b].astype(jnp.bfloat16)
        rdma1 = pltpu.make_async_remote_copy(
            src_ref=send_ref, dst_ref=recv_ref.at[0],
            send_sem=send_sems.at[0], recv_sem=recv_sems.at[0],
            device_id=(p1,), device_id_type=pl.DeviceIdType.MESH)
        rdma1.start()
        rdma1.wait()

        acc = [partials[b] + recv_ref[0, b].astype(jnp.float32)
               for b in range(B)]

        for b in range(B):
            send_ref[b] = acc[b].astype(jnp.bfloat16)
        rdma2 = pltpu.make_async_remote_copy(
            src_ref=send_ref, dst_ref=recv_ref.at[1],
            send_sem=send_sems.at[1], recv_sem=recv_sems.at[1],
            device_id=(p2,), device_id_type=pl.DeviceIdType.MESH)
        rdma2.start()
        rdma2.wait()

        for b in range(B):
            out_ref[b] = acc[b] + recv_ref[1, b].astype(jnp.float32)

    return pl.pallas_call(
        body,
        out_shape=jax.ShapeDtypeStruct((B, SQ, D), jnp.float32),
        in_specs=[pl.BlockSpec(memory_space=pltpu.VMEM)] * 5,
        out_specs=pl.BlockSpec(memory_space=pltpu.VMEM),
        scratch_shapes=[
            pltpu.VMEM((B, SQ, D), jnp.bfloat16),
            pltpu.VMEM((2, B, SQ, D), jnp.bfloat16),
            pltpu.SemaphoreType.DMA((2,)),
            pltpu.SemaphoreType.DMA((2,)),
        ],
        compiler_params=pltpu.CompilerParams(collective_id=0),
    )(x, Wq, Wo, Wk, Wv)
